# baseline (speedup 1.0000x reference)
import jax
import jax.numpy as jnp
from jax.experimental import pallas as pl
from jax.experimental.pallas import tpu as pltpu

_E = 16
_H = 1024
_I = 512
_NG = 4
_GSZ = _E // _NG
_RSF = 2.5
_EPS = 1e-20
_T = 2048
_NQ = 2
_QR = _T // _NQ


def _sig(v):
    return jax.nn.sigmoid(v)


def _moe_body(x_ref, rwt_ref, gate_ref, up_ref, down_ref, sg_ref, su_ref,
              sd_ref, out_ref, w_ref):
    e = pl.program_id(0)
    T = _T

    @pl.when(e == 0)
    def _router():
        x = x_ref[...]
        logits = jnp.dot(x, rwt_ref[...], preferred_element_type=jnp.float32)
        scores = _sig(logits)
        gsums = []
        for g in range(_NG):
            a = scores[:, 4 * g + 0:4 * g + 1]
            b = scores[:, 4 * g + 1:4 * g + 2]
            c = scores[:, 4 * g + 2:4 * g + 3]
            d = scores[:, 4 * g + 3:4 * g + 4]
            s1 = jnp.maximum(a, b); s2 = jnp.minimum(a, b)
            s3 = jnp.maximum(c, d); s4 = jnp.minimum(c, d)
            m = jnp.maximum(s1, s3)
            sec = jnp.maximum(jnp.minimum(s1, s3), jnp.maximum(s2, s4))
            gsums.append(m + sec)
        gs = jnp.concatenate(gsums, axis=1)
        cidx = jax.lax.broadcasted_iota(jnp.int32, (T, _NG), 1)
        m1 = jnp.max(gs, axis=1, keepdims=True)
        i1 = jnp.min(jnp.where(gs == m1, cidx, 9), axis=1, keepdims=True)
        e1 = cidx == i1
        gs2 = jnp.where(e1, -jnp.inf, gs)
        m2 = jnp.max(gs2, axis=1, keepdims=True)
        i2 = jnp.min(jnp.where(gs2 == m2, cidx, 9), axis=1, keepdims=True)
        gmask = jnp.logical_or(e1, cidx == i2).astype(jnp.float32)
        emask = jnp.concatenate(
            [jnp.broadcast_to(gmask[:, g:g + 1], (T, _GSZ))
             for g in range(_NG)], axis=1)
        masked = scores * emask
        denom = jnp.sum(masked, axis=1, keepdims=True)
        w_ref[...] = masked / (denom + _EPS) * _RSF
        out_ref[...] = jnp.zeros_like(out_ref)

    @pl.when(e < _E // 2)
    def _routed():
        i2 = jax.lax.broadcasted_iota(jnp.int32, (_E, 2), 0)
        c2 = jax.lax.broadcasted_iota(jnp.int32, (_E, 2), 1)
        onehot = (i2 == 2 * e + c2).astype(jnp.float32)
        wc = jnp.dot(w_ref[:, 0:_E], onehot, preferred_element_type=jnp.float32)
        for q in range(_NQ):
            rows = pl.ds(q * _QR, _QR)
            xq = x_ref[rows, :]
            acc = None
            for j in range(2):
                gq = jnp.dot(xq, gate_ref[j], preferred_element_type=jnp.float32)
                uq = jnp.dot(xq, up_ref[j], preferred_element_type=jnp.float32)
                hq = (uq * wc[q * _QR:(q + 1) * _QR, j:j + 1]) * (gq * _sig(gq))
                dq = jnp.dot(hq, down_ref[j], preferred_element_type=jnp.float32)
                acc = dq if acc is None else acc + dq
            out_ref[rows, :] += acc

    @pl.when(e == _E // 2)
    def _sharedexp():
        for q in range(_NQ):
            rows = pl.ds(q * _QR, _QR)
            xq = x_ref[rows, :]
            gq = jnp.dot(xq, sg_ref[...], preferred_element_type=jnp.float32)
            uq = jnp.dot(xq, su_ref[...], preferred_element_type=jnp.float32)
            hq = uq * (gq * _sig(gq))
            out_ref[rows, :] += jnp.dot(hq, sd_ref[...],
                                        preferred_element_type=jnp.float32)


def kernel(hidden_states, router_w, gate_w, up_w, down_w, shared_gate_w,
           shared_up_w, shared_down_w):
    B, S, Hd = hidden_states.shape
    x = hidden_states.reshape(_T, Hd)
    rwt = router_w.T

    out = pl.pallas_call(
        _moe_body,
        grid=(_E // 2 + 1,),
        in_specs=[
            pl.BlockSpec((_T, _H), lambda e: (0, 0)),
            pl.BlockSpec((_H, _E), lambda e: (0, 0)),
            pl.BlockSpec((2, _H, _I), lambda e: (jnp.minimum(e, _E // 2 - 1), 0, 0)),
            pl.BlockSpec((2, _H, _I), lambda e: (jnp.minimum(e, _E // 2 - 1), 0, 0)),
            pl.BlockSpec((2, _I, _H), lambda e: (jnp.minimum(e, _E // 2 - 1), 0, 0)),
            pl.BlockSpec((_H, _I), lambda e: (0, 0)),
            pl.BlockSpec((_H, _I), lambda e: (0, 0)),
            pl.BlockSpec((_I, _H), lambda e: (0, 0)),
        ],
        out_specs=pl.BlockSpec((_T, _H), lambda e: (0, 0)),
        out_shape=jax.ShapeDtypeStruct((_T, _H), jnp.float32),
        scratch_shapes=[pltpu.VMEM((_T, _E), jnp.float32)],
        compiler_params=pltpu.CompilerParams(
            dimension_semantics=("arbitrary",)),
    )(x, rwt, gate_w, up_w, down_w, shared_gate_w, shared_up_w, shared_down_w)
    return out.reshape(B, S, Hd)


# speedup vs baseline: 1.0108x; 1.0108x over previous
import jax
import jax.numpy as jnp
from jax.experimental import pallas as pl
from jax.experimental.pallas import tpu as pltpu

_E = 16
_H = 1024
_I = 512
_NG = 4
_GSZ = _E // _NG
_RSF = 2.5
_EPS = 1e-20
_T = 2048
_NQ = 2
_QR = _T // _NQ


def _sig(v):
    return jax.nn.sigmoid(v)


def _moe_body(x_ref, rwt_ref, gate_ref, up_ref, down_ref, sg_ref, su_ref,
              sd_ref, out_ref, w_ref):
    e = pl.program_id(0)
    T = _T

    @pl.when(e == 0)
    def _router():
        x = x_ref[...]
        logits = jnp.dot(x, rwt_ref[...], preferred_element_type=jnp.float32)
        scores = _sig(logits)
        gsums = []
        for g in range(_NG):
            a = scores[:, 4 * g + 0:4 * g + 1]
            b = scores[:, 4 * g + 1:4 * g + 2]
            c = scores[:, 4 * g + 2:4 * g + 3]
            d = scores[:, 4 * g + 3:4 * g + 4]
            s1 = jnp.maximum(a, b); s2 = jnp.minimum(a, b)
            s3 = jnp.maximum(c, d); s4 = jnp.minimum(c, d)
            m = jnp.maximum(s1, s3)
            sec = jnp.maximum(jnp.minimum(s1, s3), jnp.maximum(s2, s4))
            gsums.append(m + sec)
        gs = jnp.concatenate(gsums, axis=1)
        cidx = jax.lax.broadcasted_iota(jnp.int32, (T, _NG), 1)
        m1 = jnp.max(gs, axis=1, keepdims=True)
        i1 = jnp.min(jnp.where(gs == m1, cidx, 9), axis=1, keepdims=True)
        e1 = cidx == i1
        gs2 = jnp.where(e1, -jnp.inf, gs)
        m2 = jnp.max(gs2, axis=1, keepdims=True)
        i2 = jnp.min(jnp.where(gs2 == m2, cidx, 9), axis=1, keepdims=True)
        gmask = jnp.logical_or(e1, cidx == i2).astype(jnp.float32)
        emask = jnp.concatenate(
            [jnp.broadcast_to(gmask[:, g:g + 1], (T, _GSZ))
             for g in range(_NG)], axis=1)
        masked = scores * emask
        denom = jnp.sum(masked, axis=1, keepdims=True)
        w_ref[...] = masked / (denom + _EPS) * _RSF
        out_ref[...] = jnp.zeros_like(out_ref)

    @pl.when(e < _E)
    def _routed():
        onehot = (jax.lax.broadcasted_iota(jnp.int32, (_E, 1), 0)
                  == e).astype(jnp.float32)
        wcol = jnp.dot(w_ref[...], onehot, preferred_element_type=jnp.float32)
        dwb = down_ref[0].astype(jnp.bfloat16)
        for q in range(_NQ):
            rows = pl.ds(q * _QR, _QR)
            xq = x_ref[rows, :]
            gq = jnp.dot(xq, gate_ref[0], preferred_element_type=jnp.float32)
            uq = jnp.dot(xq, up_ref[0], preferred_element_type=jnp.float32)
            hq = ((uq * wcol[q * _QR:(q + 1) * _QR, :])
                  * (gq * _sig(gq))).astype(jnp.bfloat16)
            out_ref[rows, :] += jnp.dot(hq, dwb,
                                        preferred_element_type=jnp.float32)

    @pl.when(e == _E)
    def _sharedexp():
        for q in range(_NQ):
            rows = pl.ds(q * _QR, _QR)
            xq = x_ref[rows, :]
            gq = jnp.dot(xq, sg_ref[...], preferred_element_type=jnp.float32)
            uq = jnp.dot(xq, su_ref[...], preferred_element_type=jnp.float32)
            hq = uq * (gq * _sig(gq))
            out_ref[rows, :] += jnp.dot(hq, sd_ref[...],
                                        preferred_element_type=jnp.float32)


def kernel(hidden_states, router_w, gate_w, up_w, down_w, shared_gate_w,
           shared_up_w, shared_down_w):
    B, S, Hd = hidden_states.shape
    x = hidden_states.reshape(_T, Hd)
    rwt = router_w.T

    out = pl.pallas_call(
        _moe_body,
        grid=(_E + 1,),
        in_specs=[
            pl.BlockSpec((_T, _H), lambda e: (0, 0)),
            pl.BlockSpec((_H, _E), lambda e: (0, 0)),
            pl.BlockSpec((1, _H, _I), lambda e: (jnp.minimum(e, _E - 1), 0, 0)),
            pl.BlockSpec((1, _H, _I), lambda e: (jnp.minimum(e, _E - 1), 0, 0)),
            pl.BlockSpec((1, _I, _H), lambda e: (jnp.minimum(e, _E - 1), 0, 0)),
            pl.BlockSpec((_H, _I), lambda e: (0, 0)),
            pl.BlockSpec((_H, _I), lambda e: (0, 0)),
            pl.BlockSpec((_I, _H), lambda e: (0, 0)),
        ],
        out_specs=pl.BlockSpec((_T, _H), lambda e: (0, 0)),
        out_shape=jax.ShapeDtypeStruct((_T, _H), jnp.float32),
        scratch_shapes=[pltpu.VMEM((_T, _E), jnp.float32)],
        compiler_params=pltpu.CompilerParams(
            dimension_semantics=("arbitrary",)),
    )(x, rwt, gate_w, up_w, down_w, shared_gate_w, shared_up_w, shared_down_w)
    return out.reshape(B, S, Hd)


# speedup vs baseline: 1.0126x; 1.0018x over previous
import jax
import jax.numpy as jnp
from jax.experimental import pallas as pl
from jax.experimental.pallas import tpu as pltpu

_E = 16
_H = 1024
_I = 512
_NG = 4
_GSZ = _E // _NG
_RSF = 2.5
_EPS = 1e-20
_T = 2048
_NQ = 2
_QR = _T // _NQ


def _sig(v):
    return jax.nn.sigmoid(v)


def _moe_body(x_ref, rwt_ref, gate_ref, up_ref, down_ref, sg_ref, su_ref,
              sd_ref, out_ref, w_ref):
    e = pl.program_id(0)
    T = _T

    @pl.when(e == 0)
    def _router():
        x = x_ref[...]
        logits = jnp.dot(x, rwt_ref[...], preferred_element_type=jnp.float32)
        scores = _sig(logits)
        gsums = []
        for g in range(_NG):
            a = scores[:, 4 * g + 0:4 * g + 1]
            b = scores[:, 4 * g + 1:4 * g + 2]
            c = scores[:, 4 * g + 2:4 * g + 3]
            d = scores[:, 4 * g + 3:4 * g + 4]
            s1 = jnp.maximum(a, b); s2 = jnp.minimum(a, b)
            s3 = jnp.maximum(c, d); s4 = jnp.minimum(c, d)
            m = jnp.maximum(s1, s3)
            sec = jnp.maximum(jnp.minimum(s1, s3), jnp.maximum(s2, s4))
            gsums.append(m + sec)
        gs = jnp.concatenate(gsums, axis=1)
        cidx = jax.lax.broadcasted_iota(jnp.int32, (T, _NG), 1)
        m1 = jnp.max(gs, axis=1, keepdims=True)
        i1 = jnp.min(jnp.where(gs == m1, cidx, 9), axis=1, keepdims=True)
        e1 = cidx == i1
        gs2 = jnp.where(e1, -jnp.inf, gs)
        m2 = jnp.max(gs2, axis=1, keepdims=True)
        i2 = jnp.min(jnp.where(gs2 == m2, cidx, 9), axis=1, keepdims=True)
        gmask = jnp.logical_or(e1, cidx == i2).astype(jnp.float32)
        emask = jnp.concatenate(
            [jnp.broadcast_to(gmask[:, g:g + 1], (T, _GSZ))
             for g in range(_NG)], axis=1)
        masked = scores * emask
        denom = jnp.sum(masked, axis=1, keepdims=True)
        w_ref[...] = masked / (denom + _EPS) * _RSF
        out_ref[...] = jnp.zeros_like(out_ref)

    @pl.when(e < _E)
    def _routed():
        onehot = (jax.lax.broadcasted_iota(jnp.int32, (_E, 1), 0)
                  == e).astype(jnp.float32)
        wcol = jnp.dot(w_ref[...], onehot, preferred_element_type=jnp.float32)
        for q in range(_NQ):
            rows = pl.ds(q * _QR, _QR)
            xq = x_ref[rows, :]
            gq = jnp.dot(xq, gate_ref[0], preferred_element_type=jnp.float32)
            uq = jnp.dot(xq, up_ref[0], preferred_element_type=jnp.float32)
            hq = (uq * wcol[q * _QR:(q + 1) * _QR, :]) * (gq * _sig(gq))
            out_ref[rows, :] += jnp.dot(hq, down_ref[0],
                                        preferred_element_type=jnp.float32)

    @pl.when(e == _E)
    def _sharedexp():
        for q in range(_NQ):
            rows = pl.ds(q * _QR, _QR)
            xq = x_ref[rows, :]
            gq = jnp.dot(xq, sg_ref[...], preferred_element_type=jnp.float32)
            uq = jnp.dot(xq, su_ref[...], preferred_element_type=jnp.float32)
            hq = uq * (gq * _sig(gq))
            out_ref[rows, :] += jnp.dot(hq, sd_ref[...],
                                        preferred_element_type=jnp.float32)


def kernel(hidden_states, router_w, gate_w, up_w, down_w, shared_gate_w,
           shared_up_w, shared_down_w):
    B, S, Hd = hidden_states.shape
    x = hidden_states.reshape(_T, Hd)
    rwt = router_w.T

    out = pl.pallas_call(
        _moe_body,
        grid=(_E + 1,),
        in_specs=[
            pl.BlockSpec((_T, _H), lambda e: (0, 0)),
            pl.BlockSpec((_H, _E), lambda e: (0, 0)),
            pl.BlockSpec((1, _H, _I), lambda e: (jnp.minimum(e, _E - 1), 0, 0)),
            pl.BlockSpec((1, _H, _I), lambda e: (jnp.minimum(e, _E - 1), 0, 0)),
            pl.BlockSpec((1, _I, _H), lambda e: (jnp.minimum(e, _E - 1), 0, 0)),
            pl.BlockSpec((_H, _I), lambda e: (0, 0)),
            pl.BlockSpec((_H, _I), lambda e: (0, 0)),
            pl.BlockSpec((_I, _H), lambda e: (0, 0)),
        ],
        out_specs=pl.BlockSpec((_T, _H), lambda e: (0, 0)),
        out_shape=jax.ShapeDtypeStruct((_T, _H), jnp.float32),
        scratch_shapes=[pltpu.VMEM((_T, _E), jnp.float32)],
        compiler_params=pltpu.CompilerParams(
            dimension_semantics=("arbitrary",)),
    )(x, rwt, gate_w, up_w, down_w, shared_gate_w, shared_up_w, shared_down_w)
    return out.reshape(B, S, Hd)
